# Initial kernel scaffold; baseline (speedup 1.0000x reference)
import jax
import jax.numpy as jnp
from jax.experimental import pallas as pl


def kernel(x, edge_index, edge_attr, n_atoms_list, batch_idx, W_i_w, W_i_b, W_h_w, W_h_b, W_o_w, W_o_b, R1_w, R1_b, R2_w, R2_b):
    raise NotImplementedError("write your pallas kernel here")


# speedup vs baseline: 1.1442x; 1.1442x over previous
import functools

import jax
import jax.numpy as jnp
from jax import lax
from jax.experimental import pallas as pl
from jax.experimental.pallas import tpu as pltpu
from jax.experimental.pallas import tpu_sc as plsc

F32 = jnp.float32

_N = 100000
_E = 1600000
_H = 64
_B = 1000

_E_PAD = 1638400
_N_PAD = 100352
_RANGE = 25088
_ACC_ROWS = _RANGE + 32
_NC, _NS = 2, 16
_NW = _NC * _NS


def _mesh():
    return plsc.VectorSubcoreMesh(core_axis_name="c", subcore_axis_name="s")


def _sc_gather(table, idx2d, D, M):
    per_w = M // _NW
    chunks = per_w // 512

    @functools.partial(
        pl.kernel,
        out_type=jax.ShapeDtypeStruct((M, D), F32),
        mesh=_mesh(),
        scratch_types=[
            pltpu.VMEM((4, 128), jnp.int32),
            pltpu.VMEM((512, D), F32),
            pltpu.SemaphoreType.DMA,
        ],
        compiler_params=pltpu.CompilerParams(use_tc_tiling_on_sc=False),
    )
    def k(table_hbm, idx_hbm, out_hbm, idxv, rows, sem):
        c = lax.axis_index("c")
        s = lax.axis_index("s")
        wid = s * _NC + c
        base = wid * per_w
        ibase = wid * (per_w // 128)

        def chunk(j, carry):
            off = base + j * 512
            pltpu.sync_copy(idx_hbm.at[pl.ds(ibase + j * 4, 4)], idxv)
            cps = [
                pltpu.async_copy(
                    table_hbm.at[idxv.at[jb]],
                    rows.at[pl.ds(jb * 128, 128)],
                    sem,
                )
                for jb in range(4)
            ]
            for cp in cps:
                cp.wait()
            pltpu.sync_copy(rows, out_hbm.at[pl.ds(off, 512)])
            return carry

        lax.fori_loop(0, chunks, chunk, 0)

    return k(table, idx2d)


def _sc_segsum(h, dst2d, zeros128):
    per_tile = _E_PAD // _NS
    chunks = per_tile // 256
    zrows = _RANGE // _NS

    @functools.partial(
        pl.kernel,
        out_type=jax.ShapeDtypeStruct((_N_PAD, _H), F32),
        mesh=_mesh(),
        scratch_types=[
            pltpu.VMEM((256, _H), F32),
            pltpu.VMEM((2, 128), jnp.int32),
            pltpu.VMEM((2, 128), jnp.int32),
            pltpu.VMEM((128, _H), F32),
            pltpu.VMEM_SHARED((_ACC_ROWS, _H), F32),
        ],
        compiler_params=pltpu.CompilerParams(use_tc_tiling_on_sc=False),
    )
    def k(h_hbm, dst_hbm, z_hbm, msg_hbm, hbuf, dbuf, ibuf, zbuf, acc):
        c = lax.axis_index("c")
        s = lax.axis_index("s")
        pltpu.sync_copy(z_hbm, zbuf)
        row0 = s * zrows

        for rnd in range(2):
            base = (2 * rnd + c) * _RANGE
            for t in range(12):
                pltpu.sync_copy(zbuf, acc.at[pl.ds(row0 + t * 128, 128)])
            pltpu.sync_copy(zbuf.at[pl.ds(0, 32)],
                            acc.at[pl.ds(row0 + 12 * 128, 32)])
            plsc.subcore_barrier()

            def chunk(j, carry):
                r0 = s * per_tile + j * 256
                pltpu.sync_copy(h_hbm.at[pl.ds(r0, 256)], hbuf)
                pltpu.sync_copy(dst_hbm.at[pl.ds(s * (per_tile // 128) + j * 2, 2)],
                                dbuf)
                for jb in range(2):
                    for m in range(8):
                        dv = dbuf[jb, pl.ds(m * 16, 16)]
                        iv = dv - base
                        ok = (iv >= 0) & (iv < _RANGE)
                        ibuf[jb, pl.ds(m * 16, 16)] = jnp.where(ok, iv, _RANGE)
                for jb in range(2):
                    pltpu.sync_copy(hbuf.at[pl.ds(jb * 128, 128)],
                                    acc.at[ibuf.at[jb]], add=True)
                return carry

            lax.fori_loop(0, chunks, chunk, 0)
            plsc.subcore_barrier()
            for t in range(7):
                pltpu.sync_copy(acc.at[pl.ds(row0 + t * 224, 224)],
                                msg_hbm.at[pl.ds(base + row0 + t * 224, 224)])
            plsc.subcore_barrier()

    return k(h, dst2d, zeros128)


def _tc_init(xg, ea, Wx, We, b):
    BE = 6400
    grid = _E_PAD // BE

    def body(xg_ref, ea_ref, wx_ref, we_ref, b_ref, o_ref):
        acc = jnp.dot(xg_ref[...], wx_ref[...], preferred_element_type=F32)
        acc = acc + jnp.dot(ea_ref[...], we_ref[...], preferred_element_type=F32)
        o_ref[...] = jnp.maximum(acc + b_ref[...], 0.0)

    return pl.pallas_call(
        body,
        grid=(grid,),
        in_specs=[
            pl.BlockSpec((BE, 16), lambda i: (i, 0)),
            pl.BlockSpec((BE, 4), lambda i: (i, 0)),
            pl.BlockSpec((16, _H), lambda i: (0, 0)),
            pl.BlockSpec((4, _H), lambda i: (0, 0)),
            pl.BlockSpec((1, _H), lambda i: (0, 0)),
        ],
        out_specs=pl.BlockSpec((BE, _H), lambda i: (i, 0)),
        out_shape=jax.ShapeDtypeStruct((_E_PAD, _H), F32),
    )(xg, ea, Wx, We, b)


def _tc_step(h, gm, W, b):
    BE = 6400
    grid = _E_PAD // BE

    def body(h_ref, gm_ref, w_ref, b_ref, o_ref):
        acc = jnp.dot(h_ref[...], w_ref[...], preferred_element_type=F32)
        o_ref[...] = jnp.maximum(acc + b_ref[...] + gm_ref[...], 0.0)

    return pl.pallas_call(
        body,
        grid=(grid,),
        in_specs=[
            pl.BlockSpec((BE, _H), lambda i: (i, 0)),
            pl.BlockSpec((BE, _H), lambda i: (i, 0)),
            pl.BlockSpec((_H, _H), lambda i: (0, 0)),
            pl.BlockSpec((1, _H), lambda i: (0, 0)),
        ],
        out_specs=pl.BlockSpec((BE, _H), lambda i: (i, 0)),
        out_shape=jax.ShapeDtypeStruct((_E_PAD, _H), F32),
    )(h, gm, W, b)


def _tc_pool(xp, hN, bidx3, Wox, Woh, bo):
    BN = 512
    grid = _N_PAD // BN

    def body(x_ref, h_ref, bi_ref, wox_ref, woh_ref, bo_ref, o_ref):
        hn = jnp.dot(x_ref[...], wox_ref[...], preferred_element_type=F32)
        hn = hn + jnp.dot(h_ref[...], woh_ref[...], preferred_element_type=F32)
        hn = jnp.maximum(hn + bo_ref[...], 0.0)
        ext = jnp.concatenate(
            [hn, jnp.ones((BN, 1), F32), jnp.zeros((BN, 63), F32)], axis=1)
        bi = bi_ref[0]
        oh = (bi == lax.broadcasted_iota(jnp.int32, (BN, 1024), 1)).astype(F32)
        contrib = lax.dot_general(oh, ext, (((0,), (0,)), ((), ())),
                                  preferred_element_type=F32)

        @pl.when(pl.program_id(0) == 0)
        def _():
            o_ref[...] = jnp.zeros_like(o_ref)

        o_ref[...] += contrib

    return pl.pallas_call(
        body,
        grid=(grid,),
        in_specs=[
            pl.BlockSpec((BN, 8), lambda i: (i, 0)),
            pl.BlockSpec((BN, _H), lambda i: (i, 0)),
            pl.BlockSpec((1, BN, 1), lambda i: (i, 0, 0)),
            pl.BlockSpec((8, _H), lambda i: (0, 0)),
            pl.BlockSpec((_H, _H), lambda i: (0, 0)),
            pl.BlockSpec((1, _H), lambda i: (0, 0)),
        ],
        out_specs=pl.BlockSpec((1024, 128), lambda i: (0, 0)),
        out_shape=jax.ShapeDtypeStruct((1024, 128), F32),
    )(xp, hN, bidx3, Wox, Woh, bo)


def _tc_head(pool, R1w, R1b, R2w, R2b):
    def body(p_ref, w1_ref, b1_ref, w2_ref, b2_ref, o_ref):
        S = p_ref[...]
        cnt = S[:, _H:_H + 1]
        gv = S[:, :_H] / jnp.maximum(cnt, 1.0)
        z = jnp.dot(gv, w1_ref[...], preferred_element_type=F32) + b1_ref[...]
        hid = 0.5 * z * (1.0 + lax.erf(z * 0.7071067811865476))
        z2 = jnp.dot(hid, w2_ref[...], preferred_element_type=F32) + b2_ref[...]
        o_ref[...] = 1.0 / (1.0 + jnp.exp(-z2))

    return pl.pallas_call(
        body,
        out_shape=jax.ShapeDtypeStruct((1024, 22), F32),
    )(pool, R1w, R1b.reshape(1, _H), R2w, R2b.reshape(1, 22))


def kernel(x, edge_index, edge_attr, n_atoms_list, batch_idx,
           W_i_w, W_i_b, W_h_w, W_h_b, W_o_w, W_o_b,
           R1_w, R1_b, R2_w, R2_b):
    assert x.shape == (_N, 8) and edge_index.shape == (2, _E)
    src = edge_index[0]
    dst = edge_index[1]

    src2d = jnp.pad(src, (0, _E_PAD - _E)).reshape(_E_PAD // 128, 128)
    dst2d = jnp.pad(dst, (0, _E_PAD - _E),
                    constant_values=1 << 20).reshape(_E_PAD // 128, 128)
    xpad16 = jnp.pad(x, ((0, 0), (0, 8)))
    ea_pad = jnp.pad(edge_attr, ((0, _E_PAD - _E), (0, 0)))
    zeros128 = jnp.zeros((128, _H), F32)

    Wx = jnp.concatenate([W_i_w[:8], jnp.zeros((8, _H), F32)], axis=0)
    We = W_i_w[8:]
    b_i = W_i_b.reshape(1, _H)
    b_h = W_h_b.reshape(1, _H)

    xg = _sc_gather(xpad16, src2d, 16, _E_PAD)
    h = _tc_init(xg, ea_pad, Wx, We, b_i)
    for _ in range(2):
        msg = _sc_segsum(h, dst2d, zeros128)
        gm = _sc_gather(msg, src2d, _H, _E_PAD)
        h = _tc_step(h, gm, W_h_w, b_h)
    hN = _sc_segsum(h, dst2d, zeros128)

    xp = jnp.pad(x, ((0, _N_PAD - _N), (0, 0)))
    bidx3 = jnp.pad(batch_idx, (0, _N_PAD - _N),
                    constant_values=_B).reshape(_N_PAD // 512, 512, 1)
    pool = _tc_pool(xp, hN, bidx3, W_o_w[:8], W_o_w[8:], W_o_b.reshape(1, _H))
    out = _tc_head(pool, R1_w, R1_b, R2_w, R2_b)
    return out[:_B]


# speedup vs baseline: 1.4383x; 1.2570x over previous
import functools

import jax
import jax.numpy as jnp
from jax import lax
from jax.experimental import pallas as pl
from jax.experimental.pallas import tpu as pltpu
from jax.experimental.pallas import tpu_sc as plsc

F32 = jnp.float32

_N = 100000
_E = 1600000
_H = 64
_B = 1000

_E_PAD = 1638400
_N_PAD = 100352
_RANGE = 25088
_ACC_ROWS = _RANGE + 32
_NC, _NS = 2, 16
_NW = _NC * _NS


def _mesh():
    return plsc.VectorSubcoreMesh(core_axis_name="c", subcore_axis_name="s")


def _sc_gather(table, idx2d, D, M):
    per_w = M // _NW
    chunks = per_w // 512

    @functools.partial(
        pl.kernel,
        out_type=jax.ShapeDtypeStruct((M, D), F32),
        mesh=_mesh(),
        scratch_types=[
            pltpu.VMEM((4, 128), jnp.int32),
            pltpu.VMEM((512, D), F32),
            pltpu.SemaphoreType.DMA,
        ],
        compiler_params=pltpu.CompilerParams(use_tc_tiling_on_sc=False),
    )
    def k(table_hbm, idx_hbm, out_hbm, idxv, rows, sem):
        c = lax.axis_index("c")
        s = lax.axis_index("s")
        wid = s * _NC + c
        base = wid * per_w
        ibase = wid * (per_w // 128)

        def chunk(j, carry):
            off = base + j * 512
            pltpu.sync_copy(idx_hbm.at[pl.ds(ibase + j * 4, 4)], idxv)
            cps = [
                pltpu.async_copy(
                    table_hbm.at[idxv.at[jb]],
                    rows.at[pl.ds(jb * 128, 128)],
                    sem,
                )
                for jb in range(4)
            ]
            for cp in cps:
                cp.wait()
            pltpu.sync_copy(rows, out_hbm.at[pl.ds(off, 512)])
            return carry

        lax.fori_loop(0, chunks, chunk, 0)

    return k(table, idx2d)


def _sc_segsum(h, dst2d, zeros128):
    per_tile = _E_PAD // _NS
    chunks = per_tile // 256
    zrows = _RANGE // _NS

    @functools.partial(
        pl.kernel,
        out_type=jax.ShapeDtypeStruct((_N_PAD, _H), F32),
        mesh=_mesh(),
        scratch_types=[
            pltpu.VMEM((256, _H), F32),
            pltpu.VMEM((2, 128), jnp.int32),
            pltpu.VMEM((2, 128), jnp.int32),
            pltpu.VMEM((128, _H), F32),
            pltpu.VMEM_SHARED((_ACC_ROWS, _H), F32),
        ],
        compiler_params=pltpu.CompilerParams(use_tc_tiling_on_sc=False),
    )
    def k(h_hbm, dst_hbm, z_hbm, msg_hbm, hbuf, dbuf, ibuf, zbuf, acc):
        c = lax.axis_index("c")
        s = lax.axis_index("s")
        pltpu.sync_copy(z_hbm, zbuf)
        row0 = s * zrows

        for rnd in range(2):
            base = (2 * rnd + c) * _RANGE
            for t in range(12):
                pltpu.sync_copy(zbuf, acc.at[pl.ds(row0 + t * 128, 128)])
            pltpu.sync_copy(zbuf.at[pl.ds(0, 32)],
                            acc.at[pl.ds(row0 + 12 * 128, 32)])
            plsc.subcore_barrier()

            def chunk(j, carry):
                r0 = s * per_tile + j * 256
                pltpu.sync_copy(h_hbm.at[pl.ds(r0, 256)], hbuf)
                pltpu.sync_copy(dst_hbm.at[pl.ds(s * (per_tile // 128) + j * 2, 2)],
                                dbuf)
                for jb in range(2):
                    for m in range(8):
                        dv = dbuf[jb, pl.ds(m * 16, 16)]
                        iv = dv - base
                        ok = (iv >= 0) & (iv < _RANGE)
                        ibuf[jb, pl.ds(m * 16, 16)] = jnp.where(ok, iv, _RANGE)
                for jb in range(2):
                    pltpu.sync_copy(hbuf.at[pl.ds(jb * 128, 128)],
                                    acc.at[ibuf.at[jb]], add=True)
                return carry

            lax.fori_loop(0, chunks, chunk, 0)
            plsc.subcore_barrier()
            for t in range(7):
                pltpu.sync_copy(acc.at[pl.ds(row0 + t * 224, 224)],
                                msg_hbm.at[pl.ds(base + row0 + t * 224, 224)])
            plsc.subcore_barrier()

    return k(h, dst2d, zeros128)


_E2 = _E_PAD // 2


def _tc_init(xg32, ea2, W32, W8, b2):
    BE = 3200
    grid = _E2 // BE

    def body(xg_ref, ea_ref, wx_ref, we_ref, b_ref, o_ref):
        acc = jnp.dot(xg_ref[...], wx_ref[...], preferred_element_type=F32)
        acc = acc + jnp.dot(ea_ref[...], we_ref[...], preferred_element_type=F32)
        o_ref[...] = jnp.maximum(acc + b_ref[...], 0.0)

    return pl.pallas_call(
        body,
        grid=(grid,),
        in_specs=[
            pl.BlockSpec((BE, 32), lambda i: (i, 0)),
            pl.BlockSpec((BE, 8), lambda i: (i, 0)),
            pl.BlockSpec((32, 128), lambda i: (0, 0)),
            pl.BlockSpec((8, 128), lambda i: (0, 0)),
            pl.BlockSpec((1, 128), lambda i: (0, 0)),
        ],
        out_specs=pl.BlockSpec((BE, 128), lambda i: (i, 0)),
        out_shape=jax.ShapeDtypeStruct((_E2, 128), F32),
    )(xg32, ea2, W32, W8, b2)


def _tc_step(h, gm, W2, b2):
    BE = 3200
    grid = _E2 // BE

    def body(h_ref, gm_ref, w_ref, b_ref, o_ref):
        acc = jnp.dot(h_ref[...], w_ref[...], preferred_element_type=F32)
        o_ref[...] = jnp.maximum(acc + b_ref[...] + gm_ref[...], 0.0)

    return pl.pallas_call(
        body,
        grid=(grid,),
        in_specs=[
            pl.BlockSpec((BE, 128), lambda i: (i, 0)),
            pl.BlockSpec((BE, 128), lambda i: (i, 0)),
            pl.BlockSpec((128, 128), lambda i: (0, 0)),
            pl.BlockSpec((1, 128), lambda i: (0, 0)),
        ],
        out_specs=pl.BlockSpec((BE, 128), lambda i: (i, 0)),
        out_shape=jax.ShapeDtypeStruct((_E2, 128), F32),
    )(h, gm, W2, b2)


def _tc_pool(xp, hN, bidx3, Wox, Woh, bo):
    BN = 512
    grid = _N_PAD // BN

    def body(x_ref, h_ref, bi_ref, wox_ref, woh_ref, bo_ref, o_ref):
        hn = jnp.dot(x_ref[...], wox_ref[...], preferred_element_type=F32)
        hn = hn + jnp.dot(h_ref[...], woh_ref[...], preferred_element_type=F32)
        hn = jnp.maximum(hn + bo_ref[...], 0.0)
        ext = jnp.concatenate(
            [hn, jnp.ones((BN, 1), F32), jnp.zeros((BN, 63), F32)], axis=1)
        bi = bi_ref[0]
        oh = (bi == lax.broadcasted_iota(jnp.int32, (BN, 1024), 1)).astype(F32)
        contrib = lax.dot_general(oh, ext, (((0,), (0,)), ((), ())),
                                  preferred_element_type=F32)

        @pl.when(pl.program_id(0) == 0)
        def _():
            o_ref[...] = jnp.zeros_like(o_ref)

        o_ref[...] += contrib

    return pl.pallas_call(
        body,
        grid=(grid,),
        in_specs=[
            pl.BlockSpec((BN, 8), lambda i: (i, 0)),
            pl.BlockSpec((BN, _H), lambda i: (i, 0)),
            pl.BlockSpec((1, BN, 1), lambda i: (i, 0, 0)),
            pl.BlockSpec((8, _H), lambda i: (0, 0)),
            pl.BlockSpec((_H, _H), lambda i: (0, 0)),
            pl.BlockSpec((1, _H), lambda i: (0, 0)),
        ],
        out_specs=pl.BlockSpec((1024, 128), lambda i: (0, 0)),
        out_shape=jax.ShapeDtypeStruct((1024, 128), F32),
    )(xp, hN, bidx3, Wox, Woh, bo)


def _tc_head(pool, R1w, R1b, R2w, R2b):
    def body(p_ref, w1_ref, b1_ref, w2_ref, b2_ref, o_ref):
        S = p_ref[...]
        cnt = S[:, _H:_H + 1]
        gv = S[:, :_H] / jnp.maximum(cnt, 1.0)
        z = jnp.dot(gv, w1_ref[...], preferred_element_type=F32) + b1_ref[...]
        hid = 0.5 * z * (1.0 + lax.erf(z * 0.7071067811865476))
        z2 = jnp.dot(hid, w2_ref[...], preferred_element_type=F32) + b2_ref[...]
        o_ref[...] = 1.0 / (1.0 + jnp.exp(-z2))

    return pl.pallas_call(
        body,
        out_shape=jax.ShapeDtypeStruct((1024, 22), F32),
    )(pool, R1w, R1b.reshape(1, _H), R2w, R2b.reshape(1, 22))


def kernel(x, edge_index, edge_attr, n_atoms_list, batch_idx,
           W_i_w, W_i_b, W_h_w, W_h_b, W_o_w, W_o_b,
           R1_w, R1_b, R2_w, R2_b):
    assert x.shape == (_N, 8) and edge_index.shape == (2, _E)
    src = edge_index[0]
    dst = edge_index[1]

    src2d = jnp.pad(src, (0, _E_PAD - _E)).reshape(_E_PAD // 128, 128)
    dst2d = jnp.pad(dst, (0, _E_PAD - _E),
                    constant_values=1 << 20).reshape(_E_PAD // 128, 128)
    xpad16 = jnp.pad(x, ((0, 0), (0, 8)))
    ea2 = jnp.pad(edge_attr.reshape(_E // 2, 8), ((0, _E2 - _E // 2), (0, 0)))
    zeros128 = jnp.zeros((128, _H), F32)

    z64 = jnp.zeros((_H, _H), F32)
    Wa16 = jnp.concatenate([W_i_w[:8], jnp.zeros((8, _H), F32)], axis=0)
    z16 = jnp.zeros((16, _H), F32)
    W32 = jnp.concatenate([
        jnp.concatenate([Wa16, z16], axis=1),
        jnp.concatenate([z16, Wa16], axis=1)], axis=0)
    Wb = W_i_w[8:]
    z4 = jnp.zeros((4, _H), F32)
    W8 = jnp.concatenate([
        jnp.concatenate([Wb, z4], axis=1),
        jnp.concatenate([z4, Wb], axis=1)], axis=0)
    W2 = jnp.concatenate([
        jnp.concatenate([W_h_w, z64], axis=1),
        jnp.concatenate([z64, W_h_w], axis=1)], axis=0)
    b_i2 = jnp.concatenate([W_i_b, W_i_b]).reshape(1, 128)
    b_h2 = jnp.concatenate([W_h_b, W_h_b]).reshape(1, 128)

    xg = _sc_gather(xpad16, src2d, 16, _E_PAD)
    h = _tc_init(xg.reshape(_E2, 32), ea2, W32, W8, b_i2)
    for _ in range(2):
        msg = _sc_segsum(h.reshape(_E_PAD, _H), dst2d, zeros128)
        gm = _sc_gather(msg, src2d, _H, _E_PAD)
        h = _tc_step(h, gm.reshape(_E2, 128), W2, b_h2)
    hN = _sc_segsum(h.reshape(_E_PAD, _H), dst2d, zeros128)

    xp = jnp.pad(x, ((0, _N_PAD - _N), (0, 0)))
    bidx3 = jnp.pad(batch_idx, (0, _N_PAD - _N),
                    constant_values=_B).reshape(_N_PAD // 512, 512, 1)
    pool = _tc_pool(xp, hN, bidx3, W_o_w[:8], W_o_w[8:], W_o_b.reshape(1, _H))
    out = _tc_head(pool, R1_w, R1_b, R2_w, R2_b)
    return out[:_B]


# speedup vs baseline: 2.4730x; 1.7194x over previous
import functools

import jax
import jax.numpy as jnp
from jax import lax
from jax.experimental import pallas as pl
from jax.experimental.pallas import tpu as pltpu
from jax.experimental.pallas import tpu_sc as plsc

F32 = jnp.float32

_N = 100000
_E = 1600000
_H = 64
_B = 1000

_E_PAD = 1638400
_N_PAD = 100352
_RANGE = 25088
_ACC_ROWS = _RANGE + 32
_NC, _NS = 2, 16
_NW = _NC * _NS


def _mesh():
    return plsc.VectorSubcoreMesh(core_axis_name="c", subcore_axis_name="s")


def _sc_gather(table, idx2d, D, M):
    per_w = M // _NW
    chunks = per_w // 512

    @functools.partial(
        pl.kernel,
        out_type=jax.ShapeDtypeStruct((M, D), F32),
        mesh=_mesh(),
        scratch_types=[
            pltpu.VMEM((4, 128), jnp.int32),
            pltpu.VMEM((512, D), F32),
            pltpu.SemaphoreType.DMA,
        ],
        compiler_params=pltpu.CompilerParams(use_tc_tiling_on_sc=False),
    )
    def k(table_hbm, idx_hbm, out_hbm, idxv, rows, sem):
        c = lax.axis_index("c")
        s = lax.axis_index("s")
        wid = s * _NC + c
        base = wid * per_w
        ibase = wid * (per_w // 128)

        def chunk(j, carry):
            off = base + j * 512
            pltpu.sync_copy(idx_hbm.at[pl.ds(ibase + j * 4, 4)], idxv)
            cps = [
                pltpu.async_copy(
                    table_hbm.at[idxv.at[jb]],
                    rows.at[pl.ds(jb * 128, 128)],
                    sem,
                )
                for jb in range(4)
            ]
            for cp in cps:
                cp.wait()
            pltpu.sync_copy(rows, out_hbm.at[pl.ds(off, 512)])
            return carry

        lax.fori_loop(0, chunks, chunk, 0)

    return k(table, idx2d)


_ACC2 = 102400
_DUMP = _ACC2 - 1
_SCH = 512


def _sc_segsum(h, dst2d, zeros16):
    per_tile = _E_PAD // _NS
    chunks = per_tile // _SCH
    ztile = _ACC2 // _NS
    ftile = _N_PAD // _NS

    @functools.partial(
        pl.kernel,
        out_type=jax.ShapeDtypeStruct((_N_PAD, _H), F32),
        mesh=_mesh(),
        scratch_types=[
            pltpu.VMEM((_SCH, 16), F32),
            pltpu.VMEM((_SCH, 16), F32),
            pltpu.VMEM((4, 128), jnp.int32),
            pltpu.VMEM((4, 128), jnp.int32),
            pltpu.VMEM((4, 128), jnp.int32),
            pltpu.VMEM((128, 16), F32),
            pltpu.VMEM_SHARED((_ACC2, 16), F32),
            pltpu.SemaphoreType.DMA,
            pltpu.SemaphoreType.DMA,
        ],
        compiler_params=pltpu.CompilerParams(use_tc_tiling_on_sc=False),
    )
    def k(h_hbm, dst_hbm, z_hbm, msg_hbm,
          hb0, hb1, db0, db1, ibuf, zbuf, acc, sem0, sem1):
        c = lax.axis_index("c")
        s = lax.axis_index("s")
        pltpu.sync_copy(z_hbm, zbuf)
        erow0 = s * per_tile
        drow0 = s * (per_tile // 128)

        def fire(j, hb, db, sem):
            pltpu.async_copy(
                h_hbm.at[pl.ds(erow0 + j * _SCH, _SCH), pl.ds(ql, 16)],
                hb, sem)
            pltpu.async_copy(dst_hbm.at[pl.ds(drow0 + j * 4, 4)], db, sem)

        def drain(hb, db, sem):
            pltpu.make_async_copy(
                h_hbm.at[pl.ds(0, _SCH), pl.ds(0, 16)], hb, sem).wait()
            pltpu.make_async_copy(dst_hbm.at[pl.ds(0, 4)], db, sem).wait()

        def consume(hb, db):
            for jb in range(4):
                for m in range(8):
                    dv = db[jb, pl.ds(m * 16, 16)]
                    ibuf[jb, pl.ds(m * 16, 16)] = jnp.minimum(dv, _DUMP)
            for jb in range(4):
                pltpu.sync_copy(hb.at[pl.ds(jb * 128, 128)],
                                acc.at[ibuf.at[jb]], add=True)

        for k_ in range(2):
            ql = c * 32 + k_ * 16

            def zero(t, carry):
                pltpu.sync_copy(zbuf, acc.at[pl.ds(s * ztile + t * 128, 128)])
                return carry
            lax.fori_loop(0, ztile // 128, zero, 0)
            plsc.subcore_barrier()

            fire(0, hb0, db0, sem0)

            def pair(t, carry):
                fire(2 * t + 1, hb1, db1, sem1)
                drain(hb0, db0, sem0)
                consume(hb0, db0)

                @pl.when(t < chunks // 2 - 1)
                def _():
                    fire(2 * t + 2, hb0, db0, sem0)
                drain(hb1, db1, sem1)
                consume(hb1, db1)
                return carry

            lax.fori_loop(0, chunks // 2, pair, 0)
            plsc.subcore_barrier()

            def flush(t, carry):
                r = s * ftile + t * 128
                pltpu.sync_copy(acc.at[pl.ds(r, 128)],
                                msg_hbm.at[pl.ds(r, 128), pl.ds(ql, 16)])
                return carry
            lax.fori_loop(0, ftile // 128, flush, 0)
            plsc.subcore_barrier()

    return k(h, dst2d, zeros16)


_E2 = _E_PAD // 2


def _tc_init(xg32, ea2, W32, W8, b2):
    BE = 3200
    grid = _E2 // BE

    def body(xg_ref, ea_ref, wx_ref, we_ref, b_ref, o_ref):
        xg = xg_ref[...]
        ea = ea_ref[...]
        wx = wx_ref[...]
        we = we_ref[...]
        lo = (jnp.dot(xg[:, :16], wx, preferred_element_type=F32)
              + jnp.dot(ea[:, :4], we, preferred_element_type=F32))
        hi = (jnp.dot(xg[:, 16:], wx, preferred_element_type=F32)
              + jnp.dot(ea[:, 4:], we, preferred_element_type=F32))
        acc = jnp.concatenate([lo, hi], axis=1)
        o_ref[...] = jnp.maximum(acc + b_ref[...], 0.0)

    return pl.pallas_call(
        body,
        grid=(grid,),
        in_specs=[
            pl.BlockSpec((BE, 32), lambda i: (i, 0)),
            pl.BlockSpec((BE, 8), lambda i: (i, 0)),
            pl.BlockSpec((16, _H), lambda i: (0, 0)),
            pl.BlockSpec((4, _H), lambda i: (0, 0)),
            pl.BlockSpec((1, 128), lambda i: (0, 0)),
        ],
        out_specs=pl.BlockSpec((BE, 128), lambda i: (i, 0)),
        out_shape=jax.ShapeDtypeStruct((_E2, 128), F32),
    )(xg32, ea2, W32, W8, b2)


def _tc_step(h, gm, W2, b2):
    BE = 3200
    grid = _E2 // BE

    def body(h_ref, gm_ref, w_ref, b_ref, o_ref):
        hv = h_ref[...]
        w = w_ref[...]
        acc = jnp.concatenate(
            [jnp.dot(hv[:, :_H], w, preferred_element_type=F32),
             jnp.dot(hv[:, _H:], w, preferred_element_type=F32)], axis=1)
        o_ref[...] = jnp.maximum(acc + b_ref[...] + gm_ref[...], 0.0)

    return pl.pallas_call(
        body,
        grid=(grid,),
        in_specs=[
            pl.BlockSpec((BE, 128), lambda i: (i, 0)),
            pl.BlockSpec((BE, 128), lambda i: (i, 0)),
            pl.BlockSpec((_H, _H), lambda i: (0, 0)),
            pl.BlockSpec((1, 128), lambda i: (0, 0)),
        ],
        out_specs=pl.BlockSpec((BE, 128), lambda i: (i, 0)),
        out_shape=jax.ShapeDtypeStruct((_E2, 128), F32),
    )(h, gm, W2, b2)


def _tc_pool(xp, hN, bidx3, Wox, Woh, bo):
    BN = 512
    grid = _N_PAD // BN

    def body(x_ref, h_ref, bi_ref, wox_ref, woh_ref, bo_ref, o_ref):
        hn = jnp.dot(x_ref[...], wox_ref[...], preferred_element_type=F32)
        hn = hn + jnp.dot(h_ref[...], woh_ref[...], preferred_element_type=F32)
        hn = jnp.maximum(hn + bo_ref[...], 0.0)
        ext = jnp.concatenate(
            [hn, jnp.ones((BN, 1), F32), jnp.zeros((BN, 63), F32)], axis=1)
        bi = bi_ref[0]
        oh = (bi == lax.broadcasted_iota(jnp.int32, (BN, 1024), 1)).astype(F32)
        contrib = lax.dot_general(oh, ext, (((0,), (0,)), ((), ())),
                                  preferred_element_type=F32,
                                  precision=lax.Precision.HIGHEST)

        @pl.when(pl.program_id(0) == 0)
        def _():
            o_ref[...] = jnp.zeros_like(o_ref)

        o_ref[...] += contrib

    return pl.pallas_call(
        body,
        grid=(grid,),
        in_specs=[
            pl.BlockSpec((BN, 8), lambda i: (i, 0)),
            pl.BlockSpec((BN, _H), lambda i: (i, 0)),
            pl.BlockSpec((1, BN, 1), lambda i: (i, 0, 0)),
            pl.BlockSpec((8, _H), lambda i: (0, 0)),
            pl.BlockSpec((_H, _H), lambda i: (0, 0)),
            pl.BlockSpec((1, _H), lambda i: (0, 0)),
        ],
        out_specs=pl.BlockSpec((1024, 128), lambda i: (0, 0)),
        out_shape=jax.ShapeDtypeStruct((1024, 128), F32),
    )(xp, hN, bidx3, Wox, Woh, bo)


def _tc_head(pool, R1w, R1b, R2w, R2b):
    def body(p_ref, w1_ref, b1_ref, w2_ref, b2_ref, o_ref):
        S = p_ref[...]
        cnt = S[:, _H:_H + 1]
        gv = S[:, :_H] / jnp.maximum(cnt, 1.0)
        z = jnp.dot(gv, w1_ref[...], preferred_element_type=F32) + b1_ref[...]
        hid = 0.5 * z * (1.0 + lax.erf(z * 0.7071067811865476))
        z2 = jnp.dot(hid, w2_ref[...], preferred_element_type=F32) + b2_ref[...]
        o_ref[...] = 1.0 / (1.0 + jnp.exp(-z2))

    return pl.pallas_call(
        body,
        out_shape=jax.ShapeDtypeStruct((1024, 22), F32),
    )(pool, R1w, R1b.reshape(1, _H), R2w, R2b.reshape(1, 22))


def kernel(x, edge_index, edge_attr, n_atoms_list, batch_idx,
           W_i_w, W_i_b, W_h_w, W_h_b, W_o_w, W_o_b,
           R1_w, R1_b, R2_w, R2_b):
    assert x.shape == (_N, 8) and edge_index.shape == (2, _E)
    src = edge_index[0]
    dst = edge_index[1]

    src2d = jnp.pad(src, (0, _E_PAD - _E)).reshape(_E_PAD // 128, 128)
    dst2d = jnp.pad(dst, (0, _E_PAD - _E),
                    constant_values=1 << 20).reshape(_E_PAD // 128, 128)
    xpad16 = jnp.pad(x, ((0, 0), (0, 8)))
    ea2 = jnp.pad(edge_attr.reshape(_E // 2, 8), ((0, _E2 - _E // 2), (0, 0)))
    zeros16 = jnp.zeros((128, 16), F32)

    Wa16 = jnp.concatenate([W_i_w[:8], jnp.zeros((8, _H), F32)], axis=0)
    Wb = W_i_w[8:]
    b_i2 = jnp.concatenate([W_i_b, W_i_b]).reshape(1, 128)
    b_h2 = jnp.concatenate([W_h_b, W_h_b]).reshape(1, 128)

    xg = _sc_gather(xpad16, src2d, 16, _E_PAD)
    h = _tc_init(xg.reshape(_E2, 32), ea2, Wa16, Wb, b_i2)
    for _ in range(2):
        msg = _sc_segsum(h.reshape(_E_PAD, _H), dst2d, zeros16)
        gm = _sc_gather(msg, src2d, _H, _E_PAD)
        h = _tc_step(h, gm.reshape(_E2, 128), W_h_w, b_h2)
    hN = _sc_segsum(h.reshape(_E_PAD, _H), dst2d, zeros16)

    xp = jnp.pad(x, ((0, _N_PAD - _N), (0, 0)))
    bidx3 = jnp.pad(batch_idx, (0, _N_PAD - _N),
                    constant_values=_B).reshape(_N_PAD // 512, 512, 1)
    pool = _tc_pool(xp, hN, bidx3, W_o_w[:8], W_o_w[8:], W_o_b.reshape(1, _H))
    out = _tc_head(pool, R1_w, R1_b, R2_w, R2_b)
    return out[:_B]


# speedup vs baseline: 2.7455x; 1.1102x over previous
import functools

import jax
import jax.numpy as jnp
from jax import lax
from jax.experimental import pallas as pl
from jax.experimental.pallas import tpu as pltpu
from jax.experimental.pallas import tpu_sc as plsc

F32 = jnp.float32

_N = 100000
_E = 1600000
_H = 64
_B = 1000

_E_PAD = 1638400
_N_PAD = 100352
_RANGE = 25088
_ACC_ROWS = _RANGE + 32
_NC, _NS = 2, 16
_NW = _NC * _NS


def _mesh():
    return plsc.VectorSubcoreMesh(core_axis_name="c", subcore_axis_name="s")


def _sc_gather(table, idx2d, D, M):
    per_w = M // _NW
    chunks = per_w // 512

    @functools.partial(
        pl.kernel,
        out_type=jax.ShapeDtypeStruct((M, D), F32),
        mesh=_mesh(),
        scratch_types=[
            pltpu.VMEM((4, 128), jnp.int32),
            pltpu.VMEM((512, D), F32),
            pltpu.SemaphoreType.DMA,
        ],
        compiler_params=pltpu.CompilerParams(use_tc_tiling_on_sc=False),
    )
    def k(table_hbm, idx_hbm, out_hbm, idxv, rows, sem):
        c = lax.axis_index("c")
        s = lax.axis_index("s")
        wid = s * _NC + c
        base = wid * per_w
        ibase = wid * (per_w // 128)

        def chunk(j, carry):
            off = base + j * 512
            pltpu.sync_copy(idx_hbm.at[pl.ds(ibase + j * 4, 4)], idxv)
            cps = [
                pltpu.async_copy(
                    table_hbm.at[idxv.at[jb]],
                    rows.at[pl.ds(jb * 128, 128)],
                    sem,
                )
                for jb in range(4)
            ]
            for cp in cps:
                cp.wait()
            pltpu.sync_copy(rows, out_hbm.at[pl.ds(off, 512)])
            return carry

        lax.fori_loop(0, chunks, chunk, 0)

    return k(table, idx2d)


_ACC2 = 102400
_DUMP = _ACC2 - 1
_SCH = 512


def _sc_segsum(h, dst2d, zeros16):
    per_tile = _E_PAD // _NS
    chunks = per_tile // _SCH
    ztile = _ACC2 // _NS
    ftile = _N_PAD // _NS

    @functools.partial(
        pl.kernel,
        out_type=jax.ShapeDtypeStruct((_N_PAD, _H), F32),
        mesh=_mesh(),
        scratch_types=[
            pltpu.VMEM((_SCH, 16), F32),
            pltpu.VMEM((_SCH, 16), F32),
            pltpu.VMEM((4, 128), jnp.int32),
            pltpu.VMEM((4, 128), jnp.int32),
            pltpu.VMEM((4, 128), jnp.int32),
            pltpu.VMEM((128, 16), F32),
            pltpu.VMEM_SHARED((_ACC2, 16), F32),
            pltpu.SemaphoreType.DMA,
            pltpu.SemaphoreType.DMA,
        ],
        compiler_params=pltpu.CompilerParams(use_tc_tiling_on_sc=False),
    )
    def k(h_hbm, dst_hbm, z_hbm, msg_hbm,
          hb0, hb1, db0, db1, ibuf, zbuf, acc, sem0, sem1):
        c = lax.axis_index("c")
        s = lax.axis_index("s")
        pltpu.sync_copy(z_hbm, zbuf)
        erow0 = s * per_tile
        drow0 = s * (per_tile // 128)

        def fire(j, hb, db, sem):
            pltpu.async_copy(
                h_hbm.at[pl.ds(erow0 + j * _SCH, _SCH), pl.ds(ql, 16)],
                hb, sem)
            pltpu.async_copy(dst_hbm.at[pl.ds(drow0 + j * 4, 4)], db, sem)

        def drain(hb, db, sem):
            pltpu.make_async_copy(
                h_hbm.at[pl.ds(0, _SCH), pl.ds(0, 16)], hb, sem).wait()
            pltpu.make_async_copy(dst_hbm.at[pl.ds(0, 4)], db, sem).wait()

        def consume(hb, db):
            for jb in range(4):
                for m in range(8):
                    dv = db[jb, pl.ds(m * 16, 16)]
                    ibuf[jb, pl.ds(m * 16, 16)] = jnp.minimum(dv, _DUMP)
            for jb in range(4):
                pltpu.sync_copy(hb.at[pl.ds(jb * 128, 128)],
                                acc.at[ibuf.at[jb]], add=True)

        for k_ in range(2):
            ql = c * 32 + k_ * 16

            def zero(t, carry):
                pltpu.sync_copy(zbuf, acc.at[pl.ds(s * ztile + t * 128, 128)])
                return carry
            lax.fori_loop(0, ztile // 128, zero, 0)
            plsc.subcore_barrier()

            fire(0, hb0, db0, sem0)

            def pair(t, carry):
                fire(2 * t + 1, hb1, db1, sem1)
                drain(hb0, db0, sem0)
                consume(hb0, db0)

                @pl.when(t < chunks // 2 - 1)
                def _():
                    fire(2 * t + 2, hb0, db0, sem0)
                drain(hb1, db1, sem1)
                consume(hb1, db1)
                return carry

            lax.fori_loop(0, chunks // 2, pair, 0)
            plsc.subcore_barrier()

            def flush(t, carry):
                r = s * ftile + t * 128
                pltpu.sync_copy(acc.at[pl.ds(r, 128)],
                                msg_hbm.at[pl.ds(r, 128), pl.ds(ql, 16)])
                return carry
            lax.fori_loop(0, ftile // 128, flush, 0)
            plsc.subcore_barrier()

    return k(h, dst2d, zeros16)


_E2 = _E_PAD // 2


def _tc_init(xg32, ea_t, Wa16, Wb, b2):
    BE = 3200
    grid = _E2 // BE

    def body(xg_ref, el_ref, eh_ref, wx_ref, we_ref, b_ref, o_ref):
        xg = xg_ref[...]
        wx = wx_ref[...]
        we = we_ref[...]
        cdim = (((0,), (0,)), ((), ()))
        lo = (jnp.dot(xg[:, :16], wx, preferred_element_type=F32)
              + lax.dot_general(el_ref[...], we, cdim,
                                preferred_element_type=F32))
        hi = (jnp.dot(xg[:, 16:], wx, preferred_element_type=F32)
              + lax.dot_general(eh_ref[...], we, cdim,
                                preferred_element_type=F32))
        acc = jnp.concatenate([lo, hi], axis=1)
        o_ref[...] = jnp.maximum(acc + b_ref[...], 0.0)

    return pl.pallas_call(
        body,
        grid=(grid,),
        in_specs=[
            pl.BlockSpec((BE, 32), lambda i: (i, 0)),
            pl.BlockSpec((4, BE), lambda i: (0, i)),
            pl.BlockSpec((4, BE), lambda i: (0, i + grid)),
            pl.BlockSpec((16, _H), lambda i: (0, 0)),
            pl.BlockSpec((4, _H), lambda i: (0, 0)),
            pl.BlockSpec((1, 128), lambda i: (0, 0)),
        ],
        out_specs=pl.BlockSpec((BE, 128), lambda i: (i, 0)),
        out_shape=jax.ShapeDtypeStruct((_E2, 128), F32),
    )(xg32, ea_t, ea_t, Wa16, Wb, b2)


def _tc_step(h, gm, W2, b2):
    BE = 3200
    grid = _E2 // BE

    def body(h_ref, gm_ref, w_ref, b_ref, o_ref):
        hv = h_ref[...]
        w = w_ref[...]
        acc = jnp.concatenate(
            [jnp.dot(hv[:, :_H], w, preferred_element_type=F32),
             jnp.dot(hv[:, _H:], w, preferred_element_type=F32)], axis=1)
        o_ref[...] = jnp.maximum(acc + b_ref[...] + gm_ref[...], 0.0)

    return pl.pallas_call(
        body,
        grid=(grid,),
        in_specs=[
            pl.BlockSpec((BE, 128), lambda i: (i, 0)),
            pl.BlockSpec((BE, 128), lambda i: (i, 0)),
            pl.BlockSpec((_H, _H), lambda i: (0, 0)),
            pl.BlockSpec((1, 128), lambda i: (0, 0)),
        ],
        out_specs=pl.BlockSpec((BE, 128), lambda i: (i, 0)),
        out_shape=jax.ShapeDtypeStruct((_E2, 128), F32),
    )(h, gm, W2, b2)


def _tc_pool(xp, hN, bidx3, Wox, Woh, bo):
    BN = 512
    grid = _N_PAD // BN

    def body(x_ref, h_ref, bi_ref, wox_ref, woh_ref, bo_ref, o_ref):
        hn = jnp.dot(x_ref[...], wox_ref[...], preferred_element_type=F32)
        hn = hn + jnp.dot(h_ref[...], woh_ref[...], preferred_element_type=F32)
        hn = jnp.maximum(hn + bo_ref[...], 0.0)
        ext = jnp.concatenate(
            [hn, jnp.ones((BN, 1), F32), jnp.zeros((BN, 63), F32)], axis=1)
        bi = bi_ref[0]
        oh = (bi == lax.broadcasted_iota(jnp.int32, (BN, 1024), 1)).astype(F32)
        contrib = lax.dot_general(oh, ext, (((0,), (0,)), ((), ())),
                                  preferred_element_type=F32,
                                  precision=lax.Precision.HIGHEST)

        @pl.when(pl.program_id(0) == 0)
        def _():
            o_ref[...] = jnp.zeros_like(o_ref)

        o_ref[...] += contrib

    return pl.pallas_call(
        body,
        grid=(grid,),
        in_specs=[
            pl.BlockSpec((BN, 8), lambda i: (i, 0)),
            pl.BlockSpec((BN, _H), lambda i: (i, 0)),
            pl.BlockSpec((1, BN, 1), lambda i: (i, 0, 0)),
            pl.BlockSpec((8, _H), lambda i: (0, 0)),
            pl.BlockSpec((_H, _H), lambda i: (0, 0)),
            pl.BlockSpec((1, _H), lambda i: (0, 0)),
        ],
        out_specs=pl.BlockSpec((1024, 128), lambda i: (0, 0)),
        out_shape=jax.ShapeDtypeStruct((1024, 128), F32),
    )(xp, hN, bidx3, Wox, Woh, bo)


def _tc_head(pool, R1w, R1b, R2w, R2b):
    def body(p_ref, w1_ref, b1_ref, w2_ref, b2_ref, o_ref):
        S = p_ref[...]
        cnt = S[:, _H:_H + 1]
        gv = S[:, :_H] / jnp.maximum(cnt, 1.0)
        z = jnp.dot(gv, w1_ref[...], preferred_element_type=F32) + b1_ref[...]
        hid = 0.5 * z * (1.0 + lax.erf(z * 0.7071067811865476))
        z2 = jnp.dot(hid, w2_ref[...], preferred_element_type=F32) + b2_ref[...]
        o_ref[...] = 1.0 / (1.0 + jnp.exp(-z2))

    return pl.pallas_call(
        body,
        out_shape=jax.ShapeDtypeStruct((1024, 22), F32),
    )(pool, R1w, R1b.reshape(1, _H), R2w, R2b.reshape(1, 22))


def kernel(x, edge_index, edge_attr, n_atoms_list, batch_idx,
           W_i_w, W_i_b, W_h_w, W_h_b, W_o_w, W_o_b,
           R1_w, R1_b, R2_w, R2_b):
    assert x.shape == (_N, 8) and edge_index.shape == (2, _E)
    src = edge_index[0]
    dst = edge_index[1]

    def _interleave(v, fill):
        vp = jnp.pad(v, (0, _E_PAD - _E), constant_values=fill)
        return jnp.stack([vp[:_E2], vp[_E2:]], axis=1).reshape(_E_PAD // 128, 128)

    src2d = _interleave(src, 0)
    dst2d = _interleave(dst, 1 << 20)
    xpad16 = jnp.pad(x, ((0, 0), (0, 8)))
    ea_t = jnp.pad(edge_attr.T, ((0, 0), (0, _E_PAD - _E)))
    zeros16 = jnp.zeros((128, 16), F32)

    Wa16 = jnp.concatenate([W_i_w[:8], jnp.zeros((8, _H), F32)], axis=0)
    Wb = W_i_w[8:]
    b_i2 = jnp.concatenate([W_i_b, W_i_b]).reshape(1, 128)
    b_h2 = jnp.concatenate([W_h_b, W_h_b]).reshape(1, 128)

    xg = _sc_gather(xpad16, src2d, 16, _E_PAD)
    h = _tc_init(xg.reshape(_E2, 32), ea_t, Wa16, Wb, b_i2)
    for _ in range(2):
        msg = _sc_segsum(h.reshape(_E_PAD, _H), dst2d, zeros16)
        gm = _sc_gather(msg, src2d, _H, _E_PAD)
        h = _tc_step(h, gm.reshape(_E2, 128), W_h_w, b_h2)
    hN = _sc_segsum(h.reshape(_E_PAD, _H), dst2d, zeros16)

    xp = jnp.pad(x, ((0, _N_PAD - _N), (0, 0)))
    bidx3 = jnp.pad(batch_idx, (0, _N_PAD - _N),
                    constant_values=_B).reshape(_N_PAD // 512, 512, 1)
    pool = _tc_pool(xp, hN, bidx3, W_o_w[:8], W_o_w[8:], W_o_b.reshape(1, _H))
    out = _tc_head(pool, R1_w, R1_b, R2_w, R2_b)
    return out[:_B]
